# baseline (speedup 1.0000x reference)
import jax
import jax.numpy as jnp
from jax.experimental import pallas as pl
from jax.experimental.pallas import tpu as pltpu

_OBS = 8
_PRED = 12
_B = 256
_S = 64
_N = _B * _S
_H = 64
_M = 20
_R = 512
_G = _R // _S


def _srgat_block(xf_ref, pos_ref, nei_ref, yn_ref,
                 wte1_ref, wte2_ref, wteh_ref,
                 wq_ref, wk_ref, wv_ref, wo_ref,
                 wf1a_ref, wf1b_ref,
                 wmp_ref, p1280_ref,
                 wdd_ref, we2_ref,
                 w1a_ref, w1b_ref, w1c_ref, wdec2_ref,
                 p64_ref, p128_ref, p16_ref, pairs_ref,
                 ybest_ref, o2_ref, loss_ref):
    f32 = jnp.float32
    xf = xf_ref[...]
    p64 = p64_ref[...]
    p128 = p128_ref[...]
    p16 = p16_ref[...]

    h1 = jax.nn.relu(xf @ wte1_ref[...] + p64[0:1])
    x_enc = jax.nn.relu(h1 @ wte2_ref[...] + p64[1:2])
    hidden = jnp.tanh(h1 @ wteh_ref[...] + p64[2:3])

    vx = xf[:, 6:7]
    vy = xf[:, 13:14]
    av = vx * p128[2:3, 0:_H] + vy * p128[3:4, 0:_H]
    s_in = hidden + av
    q = s_in @ wq_ref[...]
    k = s_in @ wk_ref[...]
    v = s_in @ wv_ref[...]

    posx = pos_ref[:, 0:1]
    posy = pos_ref[:, 1:2]
    colx = jnp.broadcast_to(posx.reshape(_G, 1, _S), (_G, _S, _S)).reshape(_R, _S)
    coly = jnp.broadcast_to(posy.reshape(_G, 1, _S), (_G, _S, _S)).reshape(_R, _S)
    cx = posx - colx
    cy = posy - coly
    g16 = jax.nn.relu(cx[:, :, None] * p16[0:1, :].reshape(1, 1, 16)
                      + cy[:, :, None] * p16[1:2, :].reshape(1, 1, 16)
                      + p16[2:3, :].reshape(1, 1, 16))
    geo_rows = jnp.sum(g16 * p16[3:4, :].reshape(1, 1, 16), axis=2)

    qk = jax.lax.dot_general(q, k, (((1,), (1,)), ((), ()))) * (1.0 / 8.0)
    geo_full = jnp.concatenate([geo_rows] * _G, axis=1)
    nei = nei_ref[...]
    nei_full = jnp.concatenate([nei] * _G, axis=1)
    ri = jax.lax.broadcasted_iota(jnp.int32, (_R, _R), 0) // _S
    ci = jax.lax.broadcasted_iota(jnp.int32, (_R, _R), 1) // _S
    maskf = jnp.where(ri == ci, nei_full, 0.0)
    logits = jnp.where(maskf > 0.0, qk + geo_full, f32(-1e9))
    mx = jnp.max(logits, axis=1, keepdims=True)
    e = jnp.exp(logits - mx)
    alpha = e / jnp.sum(e, axis=1, keepdims=True) * maskf
    agg = alpha @ v
    hg = hidden + jax.nn.relu(agg @ wo_ref[...] + p64[3:4])

    gate = jax.nn.sigmoid(x_enc @ wf1a_ref[...] + hg @ wf1b_ref[...] + p64[4:5])
    fuse = gate * x_enc + (1.0 - gate) * hg
    p1280 = p1280_ref[...]
    mp = fuse @ wmp_ref[...] + p1280[2:3]
    mu = jnp.mean(mp, axis=-1, keepdims=True)
    var = jnp.mean((mp - mu) * (mp - mu), axis=-1, keepdims=True)
    mp = (mp - mu) / jnp.sqrt(var + 1e-5) * p1280[0:1] + p1280[1:2]
    mp = jax.nn.relu(mp)

    decbase = x_enc @ w1a_ref[...] + hidden @ w1b_ref[...] + p128[0:1]
    yn = yn_ref[...]
    dtx = yn[:, 22:23]
    dty = yn[:, 23:24]
    pairs = pairs_ref[...]
    wdd = wdd_ref[...]
    we2 = we2_ref[...]
    w1c = w1c_ref[...]
    wdec2 = wdec2_ref[...]

    best_l2 = None
    for m in range(_M):
        pf = mp[:, m * _H:(m + 1) * _H]
        dp = pf @ wdd + p128[6:7, 0:2]
        dpx = dp[:, 0:1]
        dpy = dp[:, 1:2]
        dn = jnp.sqrt((dpx - dtx) ** 2 + (dpy - dty) ** 2)
        e1 = jax.nn.relu(dpx * p128[4:5, 0:_H] + dpy * p128[5:6, 0:_H] + p64[5:6])
        dpf = jax.nn.relu(e1 @ we2 + p64[6:7])
        outm = jax.nn.relu(decbase + dpf @ w1c) @ wdec2 + p128[1:2, 0:24]
        d = outm - yn
        n2 = (d * d) @ pairs
        norms = jnp.sqrt(n2)
        l2m = jnp.sum(norms, axis=1, keepdims=True)
        fdem = norms[:, 11:12]
        if best_l2 is None:
            best_l2, ybest = l2m, outm
            best_fde, obest = fdem, outm
            dnmin = dn
        else:
            c1 = l2m < best_l2
            best_l2 = jnp.where(c1, l2m, best_l2)
            ybest = jnp.where(c1, outm, ybest)
            c2 = fdem < best_fde
            best_fde = jnp.where(c2, fdem, best_fde)
            obest = jnp.where(c2, outm, obest)
            dnmin = jnp.minimum(dn, dnmin)

    ybest_ref[...] = ybest
    o2_ref[...] = obest
    loss_ref[...] = jnp.concatenate([dnmin, best_l2], axis=1)


def kernel(batch_abs_gt, batch_norm_gt, nei_index, epoch, params):
    p = params
    f32 = jnp.float32
    bn = batch_norm_gt
    tx = bn[1:_OBS] - bn[0:_OBS - 1]
    xf = jnp.transpose(tx, (1, 2, 0)).reshape(_N, 2 * (_OBS - 1))
    yn = jnp.transpose(bn[_OBS:], (1, 0, 2)).reshape(_N, 2 * _PRED)
    pos = batch_abs_gt[_OBS - 1]
    nei = (nei_index > 0).reshape(_N, _S).astype(f32)

    p64 = jnp.stack([p['b_te1'], p['b_te2'], p['b_teh'], p['b_o'],
                     p['b_f1'], p['b_e1'], p['b_e2'], p['b_d1']])
    z128 = jnp.zeros((128,), f32)
    p128 = jnp.stack([
        p['b_dec1'],
        z128.at[0:24].set(p['b_dec2']),
        z128.at[0:_H].set(p['W_av'][0]),
        z128.at[0:_H].set(p['W_av'][1]),
        z128.at[0:_H].set(p['W_e1'][0]),
        z128.at[0:_H].set(p['W_e1'][1]),
        z128.at[0:2].set(p['b_d1'] @ p['W_d2'] + p['b_d2']),
        z128,
    ])
    z16 = jnp.zeros((16,), f32)
    p16 = jnp.stack([p['W_g1'][0], p['W_g1'][1], p['b_g1'],
                     p['W_g2'][:, 0], z16, z16, z16, z16])
    p1280 = jnp.concatenate([p['ln_g'][None], p['ln_b'][None],
                             p['b_mp'][None],
                             jnp.zeros((5, _M * _H), f32)], axis=0)
    w1a = p['W_dec1'][0:_H]
    w1b = p['W_dec1'][_H:2 * _H]
    w1c = p['W_dec1'][2 * _H:3 * _H]
    wf1a = p['W_f1'][0:_H]
    wf1b = p['W_f1'][_H:2 * _H]
    pairs = jnp.repeat(jnp.eye(_PRED, dtype=f32), 2, axis=0)
    wdd = p['W_d1'] @ p['W_d2']

    nblk = _N // _R
    dspec = lambda shape: pl.BlockSpec((_R, shape), lambda i: (i, 0))
    wspec = lambda a: pl.BlockSpec(a.shape, lambda i: (0,) * a.ndim)

    weights = [p['W_te1'], p['W_te2'], p['W_teh'],
               p['W_q'], p['W_k'], p['W_v'], p['W_o'],
               wf1a, wf1b,
               p['W_mp'], p1280,
               wdd, p['W_e2'],
               w1a, w1b, w1c, p['W_dec2'],
               p64, p128, p16, pairs]

    ybest24, o224, losses = pl.pallas_call(
        _srgat_block,
        grid=(nblk,),
        in_specs=[dspec(14), dspec(2), dspec(_S), dspec(2 * _PRED)]
                 + [wspec(a) for a in weights],
        out_specs=[dspec(2 * _PRED), dspec(2 * _PRED), dspec(2)],
        out_shape=[jax.ShapeDtypeStruct((_N, 2 * _PRED), f32),
                   jax.ShapeDtypeStruct((_N, 2 * _PRED), f32),
                   jax.ShapeDtypeStruct((_N, 2), f32)],
    )(xf, pos, nei, yn, *weights)

    pre_obs = bn[1:_OBS]
    y_best = ybest24.reshape(_N, _PRED, 2)
    tra1 = jnp.concatenate([pre_obs, jnp.transpose(y_best, (1, 0, 2))], axis=0)
    tra2 = jnp.concatenate(
        [pre_obs, jnp.transpose(o224.reshape(_N, _PRED, 2), (1, 0, 2))], axis=0)
    loss = jnp.mean(losses[:, 0]) + jnp.mean(losses[:, 1]) / _PRED
    return (loss, tra1, tra2)


# speedup vs baseline: 1.4499x; 1.4499x over previous
import jax
import jax.numpy as jnp
from jax.experimental import pallas as pl
from jax.experimental.pallas import tpu as pltpu

_OBS = 8
_PRED = 12
_B = 256
_S = 64
_N = _B * _S
_H = 64
_M = 20
_R = 512
_G = _R // _S


def _srgat_block(xf_ref, pos_ref, nei_ref, yn_ref, colxy_ref,
                 wte1_ref, wte2_ref, wteh_ref,
                 wq_ref, wk_ref, wv_ref, wo_ref,
                 wf1a_ref, wf1b_ref,
                 wmp_ref, p1280_ref,
                 wdd_ref, we2_ref,
                 w1a_ref, w1b_ref, w1c_ref, wdec2_ref,
                 p64_ref, p128_ref, pg_ref, ntab_ref, w2sel_ref,
                 wav_ref, we1_ref, pairs_ref,
                 ybest_ref, o2_ref, loss_ref):
    f32 = jnp.float32
    xf = xf_ref[...]
    p64 = p64_ref[...]
    p128 = p128_ref[...]

    h1 = jax.nn.relu(xf @ wte1_ref[...] + p64[0:1])
    x_enc = jax.nn.relu(h1 @ wte2_ref[...] + p64[1:2])
    hidden = jnp.tanh(h1 @ wteh_ref[...] + p64[2:3])

    vxy = jnp.concatenate([xf[:, 6:7], xf[:, 13:14]], axis=1)
    av = vxy @ wav_ref[...]
    s_in = hidden + av
    q = s_in @ wq_ref[...]
    k = s_in @ wk_ref[...]
    v = s_in @ wv_ref[...]

    pg = pg_ref[...]
    g16 = jax.nn.relu(pos_ref[...] @ pg[0:2]
                      + colxy_ref[...] @ ntab_ref[...] + pg[2:3])
    geo_rows = g16 @ w2sel_ref[...]

    qk = jax.lax.dot_general(q, k, (((1,), (1,)), ((), ()))) * (1.0 / 8.0)
    geo_full = jnp.concatenate([geo_rows] * _G, axis=1)
    nei = nei_ref[...]
    nei_full = jnp.concatenate([nei] * _G, axis=1)
    ri = jax.lax.broadcasted_iota(jnp.int32, (_R, _R), 0) // _S
    ci = jax.lax.broadcasted_iota(jnp.int32, (_R, _R), 1) // _S
    maskf = jnp.where(ri == ci, nei_full, 0.0)
    logits = jnp.where(maskf > 0.0, qk + geo_full, f32(-1e9))
    mx = jnp.max(logits, axis=1, keepdims=True)
    e = jnp.exp(logits - mx)
    alpha = e / jnp.sum(e, axis=1, keepdims=True) * maskf
    agg = alpha @ v
    hg = hidden + jax.nn.relu(agg @ wo_ref[...] + p64[3:4])

    gate = jax.nn.sigmoid(x_enc @ wf1a_ref[...] + hg @ wf1b_ref[...] + p64[4:5])
    fuse = gate * x_enc + (1.0 - gate) * hg
    p1280 = p1280_ref[...]
    mp = fuse @ wmp_ref[...] + p1280[2:3]
    mu = jnp.mean(mp, axis=-1, keepdims=True)
    var = jnp.mean((mp - mu) * (mp - mu), axis=-1, keepdims=True)
    mp = (mp - mu) / jnp.sqrt(var + 1e-5) * p1280[0:1] + p1280[1:2]
    mp = jax.nn.relu(mp)

    decbase = x_enc @ w1a_ref[...] + hidden @ w1b_ref[...] + p128[0:1]
    yn = yn_ref[...]
    dtx = yn[:, 22:23]
    dty = yn[:, 23:24]
    pairs = pairs_ref[...]
    wdd = wdd_ref[...]
    we2 = we2_ref[...]
    w1c = w1c_ref[...]
    wdec2 = wdec2_ref[...]

    best_l2 = None
    for m in range(_M):
        pf = mp[:, m * _H:(m + 1) * _H]
        dp = pf @ wdd + p128[6:7, 0:2]
        dpx = dp[:, 0:1]
        dpy = dp[:, 1:2]
        dn = jnp.sqrt((dpx - dtx) ** 2 + (dpy - dty) ** 2)
        e1 = jax.nn.relu(dp @ we1_ref[...] + p64[5:6])
        dpf = jax.nn.relu(e1 @ we2 + p64[6:7])
        outm = jax.nn.relu(decbase + dpf @ w1c) @ wdec2 + p128[1:2, 0:24]
        d = outm - yn
        n2 = (d * d) @ pairs
        norms = jnp.sqrt(n2)
        l2m = jnp.sum(norms, axis=1, keepdims=True)
        fdem = norms[:, 11:12]
        if best_l2 is None:
            best_l2, ybest = l2m, outm
            best_fde, obest = fdem, outm
            dnmin = dn
        else:
            c1 = l2m < best_l2
            best_l2 = jnp.where(c1, l2m, best_l2)
            ybest = jnp.where(c1, outm, ybest)
            c2 = fdem < best_fde
            best_fde = jnp.where(c2, fdem, best_fde)
            obest = jnp.where(c2, outm, obest)
            dnmin = jnp.minimum(dn, dnmin)

    ybest_ref[...] = ybest
    o2_ref[...] = obest
    loss_ref[...] = jnp.concatenate([dnmin, best_l2], axis=1)


def kernel(batch_abs_gt, batch_norm_gt, nei_index, epoch, params):
    p = params
    f32 = jnp.float32
    bn = batch_norm_gt
    tx = bn[1:_OBS] - bn[0:_OBS - 1]
    xf = jnp.transpose(tx, (1, 2, 0)).reshape(_N, 2 * (_OBS - 1))
    yn = jnp.transpose(bn[_OBS:], (1, 0, 2)).reshape(_N, 2 * _PRED)
    pos = batch_abs_gt[_OBS - 1]
    nei = (nei_index > 0).reshape(_N, _S).astype(f32)

    p64 = jnp.stack([p['b_te1'], p['b_te2'], p['b_teh'], p['b_o'],
                     p['b_f1'], p['b_e1'], p['b_e2'], p['b_d1']])
    z128 = jnp.zeros((128,), f32)
    p128 = jnp.stack([
        p['b_dec1'],
        z128.at[0:24].set(p['b_dec2']),
        z128.at[0:_H].set(p['W_av'][0]),
        z128.at[0:_H].set(p['W_av'][1]),
        z128.at[0:_H].set(p['W_e1'][0]),
        z128.at[0:_H].set(p['W_e1'][1]),
        z128.at[0:2].set(p['b_d1'] @ p['W_d2'] + p['b_d2']),
        z128,
    ])
    zg = jnp.zeros((16 * _S,), f32)
    pg = jnp.stack([jnp.repeat(p['W_g1'][0], _S), jnp.repeat(p['W_g1'][1], _S),
                    jnp.repeat(p['b_g1'], _S), zg, zg, zg, zg, zg])
    eye_s = jnp.eye(_S, dtype=f32)
    ntab = -jnp.concatenate([jnp.kron(p['W_g1'][0:1], eye_s),
                             jnp.kron(p['W_g1'][1:2], eye_s)], axis=0)
    w2sel = jnp.kron(p['W_g2'], eye_s)
    pxs = pos[:, 0].reshape(_B, _S)
    pys = pos[:, 1].reshape(_B, _S)
    colxy = jnp.concatenate(
        [jnp.broadcast_to(pxs[:, None, :], (_B, _S, _S)).reshape(_N, _S),
         jnp.broadcast_to(pys[:, None, :], (_B, _S, _S)).reshape(_N, _S)],
        axis=1)
    p1280 = jnp.concatenate([p['ln_g'][None], p['ln_b'][None],
                             p['b_mp'][None],
                             jnp.zeros((5, _M * _H), f32)], axis=0)
    w1a = p['W_dec1'][0:_H]
    w1b = p['W_dec1'][_H:2 * _H]
    w1c = p['W_dec1'][2 * _H:3 * _H]
    wf1a = p['W_f1'][0:_H]
    wf1b = p['W_f1'][_H:2 * _H]
    pairs = jnp.repeat(jnp.eye(_PRED, dtype=f32), 2, axis=0)
    wdd = p['W_d1'] @ p['W_d2']

    nblk = _N // _R
    dspec = lambda shape: pl.BlockSpec((_R, shape), lambda i: (i, 0))
    wspec = lambda a: pl.BlockSpec(a.shape, lambda i: (0,) * a.ndim)

    weights = [p['W_te1'], p['W_te2'], p['W_teh'],
               p['W_q'], p['W_k'], p['W_v'], p['W_o'],
               wf1a, wf1b,
               p['W_mp'], p1280,
               wdd, p['W_e2'],
               w1a, w1b, w1c, p['W_dec2'],
               p64, p128, pg, ntab, w2sel,
               p['W_av'], p['W_e1'], pairs]

    ybest24, o224, losses = pl.pallas_call(
        _srgat_block,
        grid=(nblk,),
        in_specs=[dspec(14), dspec(2), dspec(_S), dspec(2 * _PRED),
                  dspec(2 * _S)]
                 + [wspec(a) for a in weights],
        out_specs=[dspec(2 * _PRED), dspec(2 * _PRED), dspec(2)],
        out_shape=[jax.ShapeDtypeStruct((_N, 2 * _PRED), f32),
                   jax.ShapeDtypeStruct((_N, 2 * _PRED), f32),
                   jax.ShapeDtypeStruct((_N, 2), f32)],
    )(xf, pos, nei, yn, colxy, *weights)

    pre_obs = bn[1:_OBS]
    y_best = ybest24.reshape(_N, _PRED, 2)
    tra1 = jnp.concatenate([pre_obs, jnp.transpose(y_best, (1, 0, 2))], axis=0)
    tra2 = jnp.concatenate(
        [pre_obs, jnp.transpose(o224.reshape(_N, _PRED, 2), (1, 0, 2))], axis=0)
    loss = jnp.mean(losses[:, 0]) + jnp.mean(losses[:, 1]) / _PRED
    return (loss, tra1, tra2)


# speedup vs baseline: 2.8101x; 1.9381x over previous
import jax
import jax.numpy as jnp
from jax.experimental import pallas as pl
from jax.experimental.pallas import tpu as pltpu

_OBS = 8
_PRED = 12
_B = 256
_S = 64
_N = _B * _S
_H = 64
_M = 20
_R = 512
_G = _R // _S


def _srgat_block(xf_ref, pos_ref, nei_ref, yn_ref, colxy_ref, vxy_ref,
                 wte1_ref, wte2_ref, wteh_ref,
                 wq_ref, wk_ref, wv_ref, wo_ref,
                 wf1a_ref, wf1b_ref,
                 wmp_ref, p1280_ref,
                 we2_ref,
                 w1a_ref, w1b_ref, w1c_ref, wdec2_ref,
                 p64_ref, p128_ref, pg_ref, ntab_ref, w2sel_ref,
                 wav_ref, wddall_ref, we1big_ref,
                 tile24_ref, pairsbig_ref, sel40_ref, lt20_ref,
                 exp24_ref, fold24_ref,
                 ybest_ref, o2_ref, loss_ref):
    f32 = jnp.float32
    xf = xf_ref[...]
    p64 = p64_ref[...]
    p128 = p128_ref[...]

    h1 = jax.nn.relu(xf @ wte1_ref[...] + p64[0:1])
    x_enc = jax.nn.relu(h1 @ wte2_ref[...] + p64[1:2])
    hidden = jnp.tanh(h1 @ wteh_ref[...] + p64[2:3])

    av = vxy_ref[...] @ wav_ref[...]
    s_in = hidden + av
    q = s_in @ wq_ref[...]
    k = s_in @ wk_ref[...]
    v = s_in @ wv_ref[...]

    pg = pg_ref[...]
    g16 = jax.nn.relu(pos_ref[...] @ pg[0:2]
                      + colxy_ref[...] @ ntab_ref[...] + pg[2:3])
    geo_rows = g16 @ w2sel_ref[...]

    qk = jax.lax.dot_general(q, k, (((1,), (1,)), ((), ()))) * (1.0 / 8.0)
    geo_full = jnp.concatenate([geo_rows] * _G, axis=1)
    nei = nei_ref[...]
    nei_full = jnp.concatenate([nei] * _G, axis=1)
    ri = jax.lax.broadcasted_iota(jnp.int32, (_R, _R), 0) // _S
    ci = jax.lax.broadcasted_iota(jnp.int32, (_R, _R), 1) // _S
    maskf = jnp.where(ri == ci, nei_full, 0.0)
    logits = jnp.where(maskf > 0.0, qk + geo_full, f32(-1e9))
    mx = jnp.max(logits, axis=1, keepdims=True)
    e = jnp.exp(logits - mx)
    s = jnp.sum(e, axis=1, keepdims=True)
    agg = ((e * maskf) @ v) * (1.0 / s)
    hg = hidden + jax.nn.relu(agg @ wo_ref[...] + p64[3:4])

    gate = jax.nn.sigmoid(x_enc @ wf1a_ref[...] + hg @ wf1b_ref[...] + p64[4:5])
    fuse = gate * x_enc + (1.0 - gate) * hg
    p1280 = p1280_ref[...]
    mp = fuse @ wmp_ref[...] + p1280[2:3]
    mu = jnp.mean(mp, axis=-1, keepdims=True)
    var = jnp.mean((mp - mu) * (mp - mu), axis=-1, keepdims=True)
    mp = (mp - mu) / jnp.sqrt(var + 1e-5) * p1280[0:1] + p1280[1:2]
    mp = jax.nn.relu(mp)

    decbase = x_enc @ w1a_ref[...] + hidden @ w1b_ref[...] + p128[0:1]
    yn = yn_ref[...]
    dtx = yn[:, 22:23]
    dty = yn[:, 23:24]
    we2 = we2_ref[...]
    w1c = w1c_ref[...]
    wdec2 = wdec2_ref[...]

    dpall = mp @ wddall_ref[...] + p128[6:7, 0:2 * _M]
    dxm = dpall[:, 0:_M] - dtx
    dym = dpall[:, _M:2 * _M] - dty
    dnmin = jnp.min(jnp.sqrt(dxm * dxm + dym * dym), axis=1, keepdims=True)
    e1all = jax.nn.relu(dpall @ we1big_ref[...] + p1280[4:5])

    outs = []
    for m in range(_M):
        dpf = jax.nn.relu(e1all[:, m * _H:(m + 1) * _H] @ we2 + p64[6:7])
        outs.append(jax.nn.relu(decbase + dpf @ w1c) @ wdec2)
    outall = jnp.concatenate(outs, axis=1) + p1280[3:4, 0:24 * _M]

    d = outall - yn @ tile24_ref[...]
    norms = jnp.sqrt((d * d) @ pairsbig_ref[...])
    lf = norms @ sel40_ref[...]
    l2 = lf[:, 0:_M]
    fde = lf[:, _M:2 * _M]
    best_l2 = jnp.min(l2, axis=1, keepdims=True)
    best_fde = jnp.min(fde, axis=1, keepdims=True)
    lt20 = lt20_ref[...]
    oh1 = (l2 <= best_l2).astype(f32)
    oh1 = oh1 * (oh1 @ lt20 <= 1.0).astype(f32)
    oh2 = (fde <= best_fde).astype(f32)
    oh2 = oh2 * (oh2 @ lt20 <= 1.0).astype(f32)
    exp24 = exp24_ref[...]
    fold24 = fold24_ref[...]
    ybest_ref[...] = ((oh1 @ exp24) * outall) @ fold24
    o2_ref[...] = ((oh2 @ exp24) * outall) @ fold24
    loss_ref[...] = jnp.concatenate([dnmin, best_l2], axis=1)


def kernel(batch_abs_gt, batch_norm_gt, nei_index, epoch, params):
    p = params
    f32 = jnp.float32
    bn = batch_norm_gt
    tx = bn[1:_OBS] - bn[0:_OBS - 1]
    xf = jnp.transpose(tx, (1, 2, 0)).reshape(_N, 2 * (_OBS - 1))
    yn = jnp.transpose(bn[_OBS:], (1, 0, 2)).reshape(_N, 2 * _PRED)
    pos = batch_abs_gt[_OBS - 1]
    nei = (nei_index > 0).reshape(_N, _S).astype(f32)

    p64 = jnp.stack([p['b_te1'], p['b_te2'], p['b_teh'], p['b_o'],
                     p['b_f1'], p['b_e1'], p['b_e2'], p['b_d1']])
    z128 = jnp.zeros((128,), f32)
    bdd = p['b_d1'] @ p['W_d2'] + p['b_d2']
    p128 = jnp.stack([
        p['b_dec1'],
        z128,
        z128,
        z128,
        z128,
        z128,
        z128.at[0:2 * _M].set(jnp.concatenate(
            [jnp.full((_M,), bdd[0], f32), jnp.full((_M,), bdd[1], f32)])),
        z128,
    ])
    zg = jnp.zeros((16 * _S,), f32)
    pg = jnp.stack([jnp.repeat(p['W_g1'][0], _S), jnp.repeat(p['W_g1'][1], _S),
                    jnp.repeat(p['b_g1'], _S), zg, zg, zg, zg, zg])
    eye_s = jnp.eye(_S, dtype=f32)
    ntab = -jnp.concatenate([jnp.kron(p['W_g1'][0:1], eye_s),
                             jnp.kron(p['W_g1'][1:2], eye_s)], axis=0)
    w2sel = jnp.kron(p['W_g2'], eye_s)
    pxs = pos[:, 0].reshape(_B, _S)
    pys = pos[:, 1].reshape(_B, _S)
    colxy = jnp.concatenate(
        [jnp.broadcast_to(pxs[:, None, :], (_B, _S, _S)).reshape(_N, _S),
         jnp.broadcast_to(pys[:, None, :], (_B, _S, _S)).reshape(_N, _S)],
        axis=1)
    zmh = jnp.zeros((_M * _H,), f32)
    p1280 = jnp.stack([p['ln_g'], p['ln_b'], p['b_mp'],
                       zmh.at[0:24 * _M].set(jnp.tile(p['b_dec2'], _M)),
                       jnp.tile(p['b_e1'], _M),
                       zmh, zmh, zmh])
    w1a = p['W_dec1'][0:_H]
    w1b = p['W_dec1'][_H:2 * _H]
    w1c = p['W_dec1'][2 * _H:3 * _H]
    wf1a = p['W_f1'][0:_H]
    wf1b = p['W_f1'][_H:2 * _H]
    pairs = jnp.repeat(jnp.eye(_PRED, dtype=f32), 2, axis=0)
    wdd = p['W_d1'] @ p['W_d2']
    eye_m = jnp.eye(_M, dtype=f32)
    eye24 = jnp.eye(2 * _PRED, dtype=f32)
    wddall = jnp.concatenate([jnp.kron(eye_m, wdd[:, 0:1]),
                              jnp.kron(eye_m, wdd[:, 1:2])], axis=1)
    we1big = jnp.concatenate([jnp.kron(eye_m, p['W_e1'][0:1]),
                              jnp.kron(eye_m, p['W_e1'][1:2])], axis=0)
    tile24 = jnp.tile(eye24, (1, _M))
    pairsbig = jnp.kron(eye_m, pairs)
    ones12 = jnp.ones((_PRED, 1), f32)
    e11 = jnp.zeros((_PRED, 1), f32).at[_PRED - 1, 0].set(1.0)
    sel40 = jnp.concatenate([jnp.kron(eye_m, ones12),
                             jnp.kron(eye_m, e11)], axis=1)
    lt20 = jnp.triu(jnp.ones((_M, _M), f32))
    exp24 = jnp.kron(eye_m, jnp.ones((1, 2 * _PRED), f32))
    fold24 = jnp.kron(jnp.ones((_M, 1), f32), eye24)
    vxy = xf[:, jnp.array([6, 13])]

    nblk = _N // _R
    dspec = lambda shape: pl.BlockSpec((_R, shape), lambda i: (i, 0))
    wspec = lambda a: pl.BlockSpec(a.shape, lambda i: (0,) * a.ndim)

    weights = [p['W_te1'], p['W_te2'], p['W_teh'],
               p['W_q'], p['W_k'], p['W_v'], p['W_o'],
               wf1a, wf1b,
               p['W_mp'], p1280,
               p['W_e2'],
               w1a, w1b, w1c, p['W_dec2'],
               p64, p128, pg, ntab, w2sel,
               p['W_av'], wddall, we1big,
               tile24, pairsbig, sel40, lt20, exp24, fold24]

    ybest24, o224, losses = pl.pallas_call(
        _srgat_block,
        grid=(nblk,),
        in_specs=[dspec(14), dspec(2), dspec(_S), dspec(2 * _PRED),
                  dspec(2 * _S), dspec(2)]
                 + [wspec(a) for a in weights],
        out_specs=[dspec(2 * _PRED), dspec(2 * _PRED), dspec(2)],
        out_shape=[jax.ShapeDtypeStruct((_N, 2 * _PRED), f32),
                   jax.ShapeDtypeStruct((_N, 2 * _PRED), f32),
                   jax.ShapeDtypeStruct((_N, 2), f32)],
    )(xf, pos, nei, yn, colxy, vxy, *weights)

    pre_obs = bn[1:_OBS]
    y_best = ybest24.reshape(_N, _PRED, 2)
    tra1 = jnp.concatenate([pre_obs, jnp.transpose(y_best, (1, 0, 2))], axis=0)
    tra2 = jnp.concatenate(
        [pre_obs, jnp.transpose(o224.reshape(_N, _PRED, 2), (1, 0, 2))], axis=0)
    loss = jnp.mean(losses[:, 0]) + jnp.mean(losses[:, 1]) / _PRED
    return (loss, tra1, tra2)
